# baseline (speedup 1.0000x reference)
import functools

import jax
import jax.numpy as jnp
from jax import lax
from jax.experimental import pallas as pl
from jax.experimental.pallas import tpu as pltpu
from jax.experimental.pallas import tpu_sc as plsc

_VOCAB = 100000
_EMBED_DIM = 128
_N_ROWS = 64
_STRIDE = 1562
_LANES = 16
_N_WORKERS = _N_ROWS // _LANES


def _sc_gather(x):
    mesh = plsc.VectorSubcoreMesh(
        core_axis_name="c", subcore_axis_name="s", num_cores=1
    )

    @functools.partial(
        pl.kernel,
        mesh=mesh,
        out_type=jax.ShapeDtypeStruct((_N_ROWS, _EMBED_DIM), jnp.float32),
        scratch_types=[
            pltpu.SemaphoreType.DMA,
        ],
    )
    def k(x_hbm, out_hbm, sem):
        wid = lax.axis_index("s")
        rows_per_w = _N_ROWS // 16

        copies = []
        for j in range(rows_per_w):
            b = wid * rows_per_w + j
            copies.append(
                pltpu.async_copy(x_hbm.at[b * _STRIDE], out_hbm.at[b], sem)
            )
        for c in copies:
            c.wait()

    return k(x)


def kernel(x):
    return _sc_gather(x)


# speedup vs baseline: 1.0561x; 1.0561x over previous
import functools

import jax
import jax.numpy as jnp
from jax import lax
from jax.experimental import pallas as pl
from jax.experimental.pallas import tpu as pltpu
from jax.experimental.pallas import tpu_sc as plsc

_VOCAB = 100000
_EMBED_DIM = 128
_N_ROWS = 64
_STRIDE = 1562
_LANES = 16
_N_WORKERS = _N_ROWS // _LANES


def _sc_gather(x):
    mesh = plsc.VectorSubcoreMesh(
        core_axis_name="c",
        subcore_axis_name="s",
        num_cores=1,
        num_subcores=_N_WORKERS,
    )

    @functools.partial(
        pl.kernel,
        mesh=mesh,
        out_type=jax.ShapeDtypeStruct((_N_ROWS, _EMBED_DIM), jnp.float32),
        scratch_types=[
            pltpu.VMEM((_LANES, _EMBED_DIM), jnp.float32),
            pltpu.SemaphoreType.DMA,
        ],
    )
    def k(x_hbm, out_hbm, rows_v, sem):
        wid = lax.axis_index("s")
        lanes = lax.iota(jnp.int32, _LANES)
        idx = (wid * _LANES + lanes) * _STRIDE
        pltpu.async_copy(x_hbm.at[idx], rows_v, sem).wait()
        pltpu.sync_copy(rows_v, out_hbm.at[pl.ds(wid * _LANES, _LANES)])

    return k(x)


def kernel(x):
    return _sc_gather(x)


# speedup vs baseline: 1.0785x; 1.0212x over previous
import functools

import jax
import jax.numpy as jnp
from jax import lax
from jax.experimental import pallas as pl
from jax.experimental.pallas import tpu as pltpu
from jax.experimental.pallas import tpu_sc as plsc

_VOCAB = 100000
_EMBED_DIM = 128
_N_ROWS = 64
_STRIDE = 1562
_LANES = 16
_N_WORKERS = _N_ROWS // _LANES


def _sc_gather(x):
    n_workers = 16
    rows_per_w = _N_ROWS // n_workers
    mesh = plsc.VectorSubcoreMesh(
        core_axis_name="c",
        subcore_axis_name="s",
        num_cores=1,
        num_subcores=n_workers,
    )

    @functools.partial(
        pl.kernel,
        mesh=mesh,
        out_type=jax.ShapeDtypeStruct((_N_ROWS, _EMBED_DIM), jnp.float32),
        scratch_types=[
            pltpu.VMEM((_LANES,), jnp.int32),
            pltpu.VMEM((rows_per_w, _EMBED_DIM), jnp.float32),
            pltpu.SemaphoreType.DMA,
        ],
    )
    def k(x_hbm, out_hbm, idx_v, rows_v, sem):
        wid = lax.axis_index("s")
        lanes = lax.iota(jnp.int32, _LANES)
        idx_v[...] = jnp.minimum(wid * rows_per_w + lanes, _N_ROWS - 1) * _STRIDE
        pltpu.async_copy(x_hbm.at[idx_v.at[pl.ds(0, rows_per_w)]], rows_v, sem).wait()
        pltpu.sync_copy(rows_v, out_hbm.at[pl.ds(wid * rows_per_w, rows_per_w)])

    return k(x)


def kernel(x):
    return _sc_gather(x)


# speedup vs baseline: 1.0889x; 1.0096x over previous
import functools

import jax
import jax.numpy as jnp
from jax.experimental import pallas as pl
from jax.experimental.pallas import tpu as pltpu
from jax.experimental.pallas import tpu_sc as plsc

_VOCAB = 100000
_EMBED_DIM = 128
_N_ROWS = 64
_STRIDE = 1562


def _sc_gather(x):
    mesh = plsc.ScalarSubcoreMesh(axis_name="c", num_cores=1)

    @functools.partial(
        pl.kernel,
        mesh=mesh,
        out_type=jax.ShapeDtypeStruct((_N_ROWS, _EMBED_DIM), jnp.float32),
        scratch_types=[pltpu.SemaphoreType.DMA],
    )
    def k(x_hbm, out_hbm, sem):
        copies = [
            pltpu.async_copy(x_hbm.at[b * _STRIDE], out_hbm.at[b], sem)
            for b in range(_N_ROWS)
        ]
        for c in copies:
            c.wait()

    return k(x)


def kernel(x):
    return _sc_gather(x)


# speedup vs baseline: 1.0966x; 1.0071x over previous
import functools

import jax
import jax.numpy as jnp
from jax.experimental import pallas as pl
from jax.experimental.pallas import tpu as pltpu
from jax.experimental.pallas import tpu_sc as plsc

_VOCAB = 100000
_EMBED_DIM = 128
_N_ROWS = 64
_STRIDE = 1562


def _sc_gather(x):
    mesh = plsc.ScalarSubcoreMesh(axis_name="c", num_cores=1)

    @functools.partial(
        pl.kernel,
        mesh=mesh,
        out_type=jax.ShapeDtypeStruct((_N_ROWS, _EMBED_DIM), jnp.float32),
        scratch_types=[pltpu.SemaphoreType.DMA],
    )
    def k(x_hbm, out_hbm, sem):
        for b in range(_N_ROWS):
            pltpu.async_copy(x_hbm.at[b * _STRIDE], out_hbm.at[b], sem)
        pltpu.make_async_copy(x_hbm.at[pl.ds(0, _N_ROWS)], out_hbm, sem).wait()

    return k(x)


def kernel(x):
    return _sc_gather(x)


# speedup vs baseline: 1.1060x; 1.0085x over previous
import functools

import jax
import jax.numpy as jnp
from jax.experimental import pallas as pl
from jax.experimental.pallas import tpu as pltpu
from jax.experimental.pallas import tpu_sc as plsc

_VOCAB = 100000
_EMBED_DIM = 128
_N_ROWS = 64
_STRIDE = 1562


def _sc_gather(x):
    mesh = plsc.ScalarSubcoreMesh(axis_name="c", num_cores=1)

    @functools.partial(
        pl.kernel,
        mesh=mesh,
        out_type=jax.ShapeDtypeStruct((_N_ROWS, _EMBED_DIM), jnp.float32),
        scratch_types=[pltpu.SemaphoreType.DMA],
    )
    def k(x_hbm, out_hbm, sem):
        x3 = x_hbm.at[pl.ds(0, _N_ROWS * _STRIDE)].reshape(
            8, 8 * _STRIDE, _EMBED_DIM
        )
        out3 = out_hbm.reshape(8, 8, _EMBED_DIM)
        for r in range(8):
            pltpu.async_copy(
                x3.at[:, pl.ds(r * _STRIDE, 1), :],
                out3.at[:, pl.ds(r, 1), :],
                sem,
            )
        pltpu.make_async_copy(x_hbm.at[pl.ds(0, _N_ROWS)], out_hbm, sem).wait()

    return k(x)


def kernel(x):
    return _sc_gather(x)


# speedup vs baseline: 1.1347x; 1.0259x over previous
import functools

import jax
import jax.numpy as jnp
from jax.experimental import pallas as pl
from jax.experimental.pallas import tpu as pltpu
from jax.experimental.pallas import tpu_sc as plsc

_VOCAB = 100000
_EMBED_DIM = 128
_N_ROWS = 64
_STRIDE = 1562


def _sc_gather(x):
    mesh = plsc.ScalarSubcoreMesh(axis_name="c", num_cores=1)

    @functools.partial(
        pl.kernel,
        mesh=mesh,
        out_type=jax.ShapeDtypeStruct((_N_ROWS, _EMBED_DIM), jnp.float32),
        scratch_types=[
            pltpu.VMEM_SHARED((8, 8, _EMBED_DIM), jnp.float32),
            pltpu.SemaphoreType.DMA,
        ],
    )
    def k(x_hbm, out_hbm, sp, sem):
        x3 = x_hbm.at[pl.ds(0, _N_ROWS * _STRIDE)].reshape(
            8, 8 * _STRIDE, _EMBED_DIM
        )
        for r in range(8):
            pltpu.async_copy(
                x3.at[:, pl.ds(r * _STRIDE, 1), :],
                sp.at[:, pl.ds(r, 1), :],
                sem,
            )
        pltpu.make_async_copy(x_hbm.at[pl.ds(0, _N_ROWS)], sp, sem).wait()
        pltpu.sync_copy(sp.reshape(_N_ROWS, _EMBED_DIM), out_hbm)

    return k(x)


def kernel(x):
    return _sc_gather(x)
